# baseline (speedup 1.0000x reference)
import functools

import jax
import jax.numpy as jnp
from jax import lax
from jax.experimental import pallas as pl
from jax.experimental.pallas import tpu as pltpu
from jax.experimental.pallas import tpu_sc as plsc

N = 16384
M = 4096
C_IN = 128
C_OUT = 256
K = 16
EPS = 1e-5

L = 16
NW = 32
QPW = M // NW
NCHUNK = N // L
GROUP = 16
NGROUP = NCHUNK // GROUP

_MESH = plsc.VectorSubcoreMesh(core_axis_name="c", subcore_axis_name="s")
_CP = pltpu.CompilerParams(needs_layout_passes=False)


def _round_bf16(v):
    b = plsc.bitcast(v, jnp.int32)
    b = b + 0x7FFF + ((b >> 16) & 1)
    b = b & jnp.int32(-65536)
    return plsc.bitcast(b, jnp.float32)


def _splat(vec, idx_splat):
    return vec.at[idx_splat].get(mode="promise_in_bounds")


@functools.partial(
    pl.kernel,
    mesh=_MESH,
    compiler_params=_CP,
    out_type=jax.ShapeDtypeStruct((K * M, C_IN), jnp.float32),
    scratch_types=[
        pltpu.VMEM((N,), jnp.float32),
        pltpu.VMEM((N,), jnp.float32),
        pltpu.VMEM((N,), jnp.float32),
        pltpu.VMEM((N,), jnp.float32),
        pltpu.VMEM((QPW,), jnp.float32),
        pltpu.VMEM((QPW,), jnp.float32),
        pltpu.VMEM((QPW,), jnp.float32),
        pltpu.VMEM((K, QPW), jnp.int32),
        pltpu.VMEM((QPW, C_IN), jnp.float32),
        pltpu.SemaphoreType.DMA,
    ],
)
def _knn_gather(xs_hbm, ys_hbm, zs_hbm, qx_hbm, qy_hbm, qz_hbm, feat_hbm,
                out_hbm, xs_v, ys_v, zs_v, s2_v, qx_v, qy_v, qz_v,
                idx_v, rows_v, sem):
    wid = lax.axis_index("s") * 2 + lax.axis_index("c")
    base = wid * QPW
    pltpu.sync_copy(xs_hbm, xs_v)
    pltpu.sync_copy(ys_hbm, ys_v)
    pltpu.sync_copy(zs_hbm, zs_v)
    pltpu.sync_copy(qx_hbm.at[pl.ds(base, QPW)], qx_v)
    pltpu.sync_copy(qy_hbm.at[pl.ds(base, QPW)], qy_v)
    pltpu.sync_copy(qz_hbm.at[pl.ds(base, QPW)], qz_v)
    lane = lax.iota(jnp.int32, L)
    sel15 = jnp.full((L,), L - 1, jnp.int32)
    shift_idx = jnp.maximum(lane - 1, 0)

    def prep(j, _):
        off = j * L
        x = xs_v[pl.ds(off, L)]
        y = ys_v[pl.ds(off, L)]
        z = zs_v[pl.ds(off, L)]
        s2_v[pl.ds(off, L)] = x * x + y * y + z * z
        xs_v[pl.ds(off, L)] = _round_bf16(x)
        ys_v[pl.ds(off, L)] = _round_bf16(y)
        zs_v[pl.ds(off, L)] = _round_bf16(z)
        return 0

    lax.fori_loop(0, NCHUNK, prep, 0)

    def per_pair(p, _):
        i0 = 2 * p
        g = i0 // L
        l0 = i0 % L
        sel0 = jnp.full((L,), l0, jnp.int32)
        sel1 = sel0 + 1
        qgx = qx_v[pl.ds(g * L, L)]
        qgy = qy_v[pl.ds(g * L, L)]
        qgz = qz_v[pl.ds(g * L, L)]
        axs = (_round_bf16(_splat(qgx, sel0)) * -2.0,
               _round_bf16(_splat(qgx, sel1)) * -2.0)
        ays = (_round_bf16(_splat(qgy, sel0)) * -2.0,
               _round_bf16(_splat(qgy, sel1)) * -2.0)
        azs = (_round_bf16(_splat(qgz, sel0)) * -2.0,
               _round_bf16(_splat(qgz, sel1)) * -2.0)

        def merge_chunk(td, ti, th, d, off):
            mask = d < th
            npos = plsc.all_reduce_population_count(mask)[0]

            def act(args):
                td, ti, _ = args

                def insert(a):
                    td, ti, _ = a
                    ffs = plsc.all_reduce_ffs(mask)
                    c = _splat(d, ffs)
                    ci = off + ffs
                    cnt_v = plsc.all_reduce_population_count(td <= c)
                    td_sh = _splat(td, shift_idx)
                    ti_sh = _splat(ti, shift_idx)
                    below = lane < cnt_v
                    at = lane == cnt_v
                    ntd = jnp.where(below, td, jnp.where(at, c, td_sh))
                    nti = jnp.where(below, ti, jnp.where(at, ci, ti_sh))
                    return ntd, nti, _splat(ntd, sel15)

                def full(a):
                    td, ti, _ = a
                    dd, di = plsc.sort_key_val(d, off + lane, descending=True)
                    keep = td <= dd
                    lo_d = jnp.where(keep, td, dd)
                    lo_i = jnp.where(keep, ti, di)
                    td2, ti2 = plsc.sort_key_val(lo_d, lo_i)
                    return td2, ti2, _splat(td2, sel15)

                return lax.cond(npos == 1, insert, full, (td, ti, th))

            return lax.cond(npos > 0, act, lambda a: a, (td, ti, th))

        def screen(td, ti, th, ds, off0):
            cs = [plsc.all_reduce_population_count(ds[u] < th)
                  for u in range(GROUP)]
            total = cs[0]
            for u in range(1, GROUP):
                total = total + cs[u]

            def single(a):
                td, ti, _ = a
                which = jnp.minimum(cs[0], 1) * 0
                for u in range(1, GROUP):
                    which = which + jnp.minimum(cs[u], 1) * u
                dsel = ds[0]
                for u in range(1, GROUP):
                    dsel = jnp.where(cs[u] > 0, ds[u], dsel)
                mask = dsel < th
                ffs = plsc.all_reduce_ffs(mask)
                c = _splat(dsel, ffs)
                ci = off0 + which * L + ffs
                cnt_v = plsc.all_reduce_population_count(td <= c)
                td_sh = _splat(td, shift_idx)
                ti_sh = _splat(ti, shift_idx)
                below = lane < cnt_v
                at = lane == cnt_v
                ntd = jnp.where(below, td, jnp.where(at, c, td_sh))
                nti = jnp.where(below, ti, jnp.where(at, ci, ti_sh))
                return ntd, nti, _splat(ntd, sel15)

            def multi(a):
                td, ti, th = a
                for u in range(GROUP):
                    td, ti, th = merge_chunk(td, ti, th, ds[u], off0 + u * L)
                return td, ti, th

            return lax.cond(total[0] == 1, single, multi, (td, ti, th))

        def group_step(jg, carry):
            td0, ti0, th0, td1, ti1, th1 = carry
            off0 = jg * (L * GROUP)
            d0s, d1s = [], []
            mv0 = mv1 = None
            for u in range(GROUP):
                off = off0 + u * L
                s2c = s2_v[pl.ds(off, L)]
                xc = xs_v[pl.ds(off, L)]
                yc = ys_v[pl.ds(off, L)]
                zc = zs_v[pl.ds(off, L)]
                d0 = s2c + axs[0] * xc + ays[0] * yc + azs[0] * zc
                d1 = s2c + axs[1] * xc + ays[1] * yc + azs[1] * zc
                d0s.append(d0)
                d1s.append(d1)
                mv0 = d0 if u == 0 else jnp.minimum(mv0, d0)
                mv1 = d1 if u == 0 else jnp.minimum(mv1, d1)

            def slow0(args):
                td, ti, th = args
                return screen(td, ti, th, d0s, off0)

            def slow1(args):
                td, ti, th = args
                return screen(td, ti, th, d1s, off0)

            hit0 = plsc.all_reduce_population_count(mv0 < th0)[0]
            td0, ti0, th0 = lax.cond(hit0 > 0, slow0, lambda a: a,
                                     (td0, ti0, th0))
            hit1 = plsc.all_reduce_population_count(mv1 < th1)[0]
            td1, ti1, th1 = lax.cond(hit1 > 0, slow1, lambda a: a,
                                     (td1, ti1, th1))
            return td0, ti0, th0, td1, ti1, th1

        init = (jnp.full((L,), jnp.inf, jnp.float32),
                jnp.zeros((L,), jnp.int32),
                jnp.full((L,), jnp.inf, jnp.float32)) * 2
        out = lax.fori_loop(0, NGROUP, group_step, init)
        plsc.store_scatter(idx_v, [lane, jnp.full((L,), i0, jnp.int32)],
                           out[1])
        plsc.store_scatter(idx_v, [lane, jnp.full((L,), i0 + 1, jnp.int32)],
                           out[4])
        return 0

    lax.fori_loop(0, QPW // 2, per_pair, 0)

    for k in range(K):
        pltpu.async_copy(feat_hbm.at[idx_v.at[k]], rows_v, sem).wait()
        pltpu.sync_copy(rows_v, out_hbm.at[pl.ds(k * M + base, QPW)])


BM = 256


def _tail_body(g_ref, gamma_ref, beta_ref, w_ref, o_ref):
    w = w_ref[:]
    gam = gamma_ref[:]
    bet = beta_ref[:]
    acc = jnp.full((BM, C_OUT), -jnp.inf, dtype=jnp.float32)
    for k in range(K):
        x = g_ref[k]
        mu = jnp.mean(x, axis=1, keepdims=True)
        xc = x - mu
        var = jnp.mean(xc * xc, axis=1, keepdims=True)
        xn = xc / jnp.sqrt(var + EPS) * gam + bet
        acc = jnp.maximum(acc, jnp.dot(xn, w, preferred_element_type=jnp.float32))
    o_ref[:] = acc


_tail = pl.pallas_call(
    _tail_body,
    grid=(M // BM,),
    in_specs=[
        pl.BlockSpec((K, BM, C_IN), lambda m: (0, m, 0)),
        pl.BlockSpec((1, C_IN), lambda m: (0, 0)),
        pl.BlockSpec((1, C_IN), lambda m: (0, 0)),
        pl.BlockSpec((C_IN, C_OUT), lambda m: (0, 0)),
    ],
    out_specs=pl.BlockSpec((BM, C_OUT), lambda m: (m, 0)),
    out_shape=jax.ShapeDtypeStruct((M, C_OUT), jnp.float32),
)


def kernel(src_xyz, src_feat, query_xyz, gamma, beta, W):
    s = src_xyz.T
    q = query_xyz.T
    grouped = _knn_gather(s[0], s[1], s[2], q[0], q[1], q[2], src_feat)
    g3 = grouped.reshape(K, M, C_IN)
    return _tail(g3, gamma.reshape(1, C_IN), beta.reshape(1, C_IN), W)


# speedup vs baseline: 1.3341x; 1.3341x over previous
import functools

import jax
import jax.numpy as jnp
from jax import lax
from jax.experimental import pallas as pl
from jax.experimental.pallas import tpu as pltpu
from jax.experimental.pallas import tpu_sc as plsc

N = 16384
M = 4096
C_IN = 128
C_OUT = 256
K = 16
EPS = 1e-5

L = 16
NW = 32
QPW = M // NW
NCHUNK = N // L
GROUP = 8
NGROUP = NCHUNK // GROUP

_MESH = plsc.VectorSubcoreMesh(core_axis_name="c", subcore_axis_name="s")
_CP = pltpu.CompilerParams(needs_layout_passes=False)


def _round_bf16(v):
    b = plsc.bitcast(v, jnp.int32)
    b = b + 0x7FFF + ((b >> 16) & 1)
    b = b & jnp.int32(-65536)
    return plsc.bitcast(b, jnp.float32)


def _splat(vec, idx_splat):
    return vec.at[idx_splat].get(mode="promise_in_bounds")


@functools.partial(
    pl.kernel,
    mesh=_MESH,
    compiler_params=_CP,
    out_type=jax.ShapeDtypeStruct((K * M, C_IN), jnp.float32),
    scratch_types=[
        pltpu.VMEM((N,), jnp.float32),
        pltpu.VMEM((N,), jnp.float32),
        pltpu.VMEM((N,), jnp.float32),
        pltpu.VMEM((N,), jnp.float32),
        pltpu.VMEM((QPW,), jnp.float32),
        pltpu.VMEM((QPW,), jnp.float32),
        pltpu.VMEM((QPW,), jnp.float32),
        pltpu.VMEM((K, QPW), jnp.int32),
        pltpu.VMEM((QPW, C_IN), jnp.float32),
        pltpu.SemaphoreType.DMA,
    ],
)
def _knn_gather(xs_hbm, ys_hbm, zs_hbm, qx_hbm, qy_hbm, qz_hbm, feat_hbm,
                out_hbm, xs_v, ys_v, zs_v, s2_v, qx_v, qy_v, qz_v,
                idx_v, rows_v, sem):
    wid = lax.axis_index("s") * 2 + lax.axis_index("c")
    base = wid * QPW
    pltpu.sync_copy(xs_hbm, xs_v)
    pltpu.sync_copy(ys_hbm, ys_v)
    pltpu.sync_copy(zs_hbm, zs_v)
    pltpu.sync_copy(qx_hbm.at[pl.ds(base, QPW)], qx_v)
    pltpu.sync_copy(qy_hbm.at[pl.ds(base, QPW)], qy_v)
    pltpu.sync_copy(qz_hbm.at[pl.ds(base, QPW)], qz_v)
    lane = lax.iota(jnp.int32, L)
    sel15 = jnp.full((L,), L - 1, jnp.int32)
    shift_idx = jnp.maximum(lane - 1, 0)

    def prep(j, _):
        off = j * L
        x = xs_v[pl.ds(off, L)]
        y = ys_v[pl.ds(off, L)]
        z = zs_v[pl.ds(off, L)]
        s2_v[pl.ds(off, L)] = x * x + y * y + z * z
        xs_v[pl.ds(off, L)] = _round_bf16(x)
        ys_v[pl.ds(off, L)] = _round_bf16(y)
        zs_v[pl.ds(off, L)] = _round_bf16(z)
        return 0

    lax.fori_loop(0, NCHUNK, prep, 0)

    def per_pair(p):
        i0 = 2 * p
        g = i0 // L
        l0 = i0 % L
        sel0 = jnp.full((L,), l0, jnp.int32)
        sel1 = sel0 + 1
        qgx = qx_v[pl.ds(g * L, L)]
        qgy = qy_v[pl.ds(g * L, L)]
        qgz = qz_v[pl.ds(g * L, L)]
        axs = (_round_bf16(_splat(qgx, sel0)) * -2.0,
               _round_bf16(_splat(qgx, sel1)) * -2.0)
        ays = (_round_bf16(_splat(qgy, sel0)) * -2.0,
               _round_bf16(_splat(qgy, sel1)) * -2.0)
        azs = (_round_bf16(_splat(qgz, sel0)) * -2.0,
               _round_bf16(_splat(qgz, sel1)) * -2.0)

        def merge_chunk(td, ti, th, d, off):
            mask = d < th
            npos = plsc.all_reduce_population_count(mask)[0]

            def act(args):
                td, ti, _ = args

                def insert(a):
                    td, ti, _ = a
                    ffs = plsc.all_reduce_ffs(mask)
                    c = _splat(d, ffs)
                    ci = off + ffs
                    cnt_v = plsc.all_reduce_population_count(td <= c)
                    td_sh = _splat(td, shift_idx)
                    ti_sh = _splat(ti, shift_idx)
                    below = lane < cnt_v
                    at = lane == cnt_v
                    ntd = jnp.where(below, td, jnp.where(at, c, td_sh))
                    nti = jnp.where(below, ti, jnp.where(at, ci, ti_sh))
                    return ntd, nti, _splat(ntd, sel15)

                def full(a):
                    td, ti, _ = a
                    dd, di = plsc.sort_key_val(d, off + lane, descending=True)
                    keep = td <= dd
                    lo_d = jnp.where(keep, td, dd)
                    lo_i = jnp.where(keep, ti, di)
                    td2, ti2 = plsc.sort_key_val(lo_d, lo_i)
                    return td2, ti2, _splat(td2, sel15)

                return lax.cond(npos == 1, insert, full, (td, ti, th))

            return lax.cond(npos > 0, act, lambda a: a, (td, ti, th))

        def screen(td, ti, th, ds, off0):
            cs = [plsc.all_reduce_population_count(ds[u] < th)
                  for u in range(GROUP)]
            total = cs[0]
            for u in range(1, GROUP):
                total = total + cs[u]

            def single(a):
                td, ti, _ = a
                which = jnp.minimum(cs[0], 1) * 0
                for u in range(1, GROUP):
                    which = which + jnp.minimum(cs[u], 1) * u
                dsel = ds[0]
                for u in range(1, GROUP):
                    dsel = jnp.where(cs[u] > 0, ds[u], dsel)
                mask = dsel < th
                ffs = plsc.all_reduce_ffs(mask)
                c = _splat(dsel, ffs)
                ci = off0 + which * L + ffs
                cnt_v = plsc.all_reduce_population_count(td <= c)
                td_sh = _splat(td, shift_idx)
                ti_sh = _splat(ti, shift_idx)
                below = lane < cnt_v
                at = lane == cnt_v
                ntd = jnp.where(below, td, jnp.where(at, c, td_sh))
                nti = jnp.where(below, ti, jnp.where(at, ci, ti_sh))
                return ntd, nti, _splat(ntd, sel15)

            def multi(a):
                td, ti, th = a
                for u in range(GROUP):
                    td, ti, th = merge_chunk(td, ti, th, ds[u], off0 + u * L)
                return td, ti, th

            return lax.cond(total[0] == 1, single, multi, (td, ti, th))

        def group_step(jg, carry):
            td0, ti0, th0, td1, ti1, th1 = carry
            off0 = jg * (L * GROUP)
            d0s, d1s = [], []
            mv0 = mv1 = None
            for u in range(GROUP):
                off = off0 + u * L
                s2c = s2_v[pl.ds(off, L)]
                xc = xs_v[pl.ds(off, L)]
                yc = ys_v[pl.ds(off, L)]
                zc = zs_v[pl.ds(off, L)]
                d0 = s2c + axs[0] * xc + ays[0] * yc + azs[0] * zc
                d1 = s2c + axs[1] * xc + ays[1] * yc + azs[1] * zc
                d0s.append(d0)
                d1s.append(d1)
                mv0 = d0 if u == 0 else jnp.minimum(mv0, d0)
                mv1 = d1 if u == 0 else jnp.minimum(mv1, d1)

            def slow0(args):
                td, ti, th = args
                return screen(td, ti, th, d0s, off0)

            def slow1(args):
                td, ti, th = args
                return screen(td, ti, th, d1s, off0)

            hit0 = plsc.all_reduce_population_count(mv0 < th0)[0]
            td0, ti0, th0 = lax.cond(hit0 > 0, slow0, lambda a: a,
                                     (td0, ti0, th0))
            hit1 = plsc.all_reduce_population_count(mv1 < th1)[0]
            td1, ti1, th1 = lax.cond(hit1 > 0, slow1, lambda a: a,
                                     (td1, ti1, th1))
            return td0, ti0, th0, td1, ti1, th1

        init = (jnp.full((L,), jnp.inf, jnp.float32),
                jnp.zeros((L,), jnp.int32),
                jnp.full((L,), jnp.inf, jnp.float32)) * 2
        out = lax.fori_loop(0, NGROUP, group_step, init)
        plsc.store_scatter(idx_v, [lane, jnp.full((L,), i0, jnp.int32)],
                           out[1])
        plsc.store_scatter(idx_v, [lane, jnp.full((L,), i0 + 1, jnp.int32)],
                           out[4])

    plsc.parallel_loop(0, QPW // 2, 1, unroll=2)(per_pair)

    for k in range(K):
        pltpu.async_copy(feat_hbm.at[idx_v.at[k]], rows_v, sem).wait()
        pltpu.sync_copy(rows_v, out_hbm.at[pl.ds(k * M + base, QPW)])


BM = 256


def _tail_body(g_ref, gamma_ref, beta_ref, w_ref, o_ref):
    w = w_ref[:]
    gam = gamma_ref[:]
    bet = beta_ref[:]
    acc = jnp.full((BM, C_OUT), -jnp.inf, dtype=jnp.float32)
    for k in range(K):
        x = g_ref[k]
        mu = jnp.mean(x, axis=1, keepdims=True)
        xc = x - mu
        var = jnp.mean(xc * xc, axis=1, keepdims=True)
        xn = xc / jnp.sqrt(var + EPS) * gam + bet
        acc = jnp.maximum(acc, jnp.dot(xn, w, preferred_element_type=jnp.float32))
    o_ref[:] = acc


_tail = pl.pallas_call(
    _tail_body,
    grid=(M // BM,),
    in_specs=[
        pl.BlockSpec((K, BM, C_IN), lambda m: (0, m, 0)),
        pl.BlockSpec((1, C_IN), lambda m: (0, 0)),
        pl.BlockSpec((1, C_IN), lambda m: (0, 0)),
        pl.BlockSpec((C_IN, C_OUT), lambda m: (0, 0)),
    ],
    out_specs=pl.BlockSpec((BM, C_OUT), lambda m: (m, 0)),
    out_shape=jax.ShapeDtypeStruct((M, C_OUT), jnp.float32),
)


def kernel(src_xyz, src_feat, query_xyz, gamma, beta, W):
    s = src_xyz.T
    q = query_xyz.T
    grouped = _knn_gather(s[0], s[1], s[2], q[0], q[1], q[2], src_feat)
    g3 = grouped.reshape(K, M, C_IN)
    return _tail(g3, gamma.reshape(1, C_IN), beta.reshape(1, C_IN), W)


# speedup vs baseline: 1.3568x; 1.0170x over previous
import functools

import jax
import jax.numpy as jnp
from jax import lax
from jax.experimental import pallas as pl
from jax.experimental.pallas import tpu as pltpu
from jax.experimental.pallas import tpu_sc as plsc

N = 16384
M = 4096
C_IN = 128
C_OUT = 256
K = 16
EPS = 1e-5

L = 16
NW = 32
QPW = M // NW
NCHUNK = N // L
GROUP = 8
NGROUP = NCHUNK // GROUP

_MESH = plsc.VectorSubcoreMesh(core_axis_name="c", subcore_axis_name="s")
_CP = pltpu.CompilerParams(needs_layout_passes=False)


def _round_bf16(v):
    b = plsc.bitcast(v, jnp.int32)
    b = b + 0x7FFF + ((b >> 16) & 1)
    b = b & jnp.int32(-65536)
    return plsc.bitcast(b, jnp.float32)


def _splat(vec, idx_splat):
    return vec.at[idx_splat].get(mode="promise_in_bounds")


@functools.partial(
    pl.kernel,
    mesh=_MESH,
    compiler_params=_CP,
    out_type=jax.ShapeDtypeStruct((K * M, C_IN), jnp.float32),
    scratch_types=[
        pltpu.VMEM((N,), jnp.float32),
        pltpu.VMEM((N,), jnp.float32),
        pltpu.VMEM((N,), jnp.float32),
        pltpu.VMEM((N,), jnp.float32),
        pltpu.VMEM((QPW,), jnp.float32),
        pltpu.VMEM((QPW,), jnp.float32),
        pltpu.VMEM((QPW,), jnp.float32),
        pltpu.VMEM((K, QPW), jnp.int32),
        pltpu.VMEM((QPW, C_IN), jnp.float32),
        pltpu.SemaphoreType.DMA,
    ],
)
def _knn_gather(xs_hbm, ys_hbm, zs_hbm, qx_hbm, qy_hbm, qz_hbm, feat_hbm,
                out_hbm, xs_v, ys_v, zs_v, s2_v, qx_v, qy_v, qz_v,
                idx_v, rows_v, sem):
    wid = lax.axis_index("s") * 2 + lax.axis_index("c")
    base = wid * QPW
    pltpu.sync_copy(xs_hbm, xs_v)
    pltpu.sync_copy(ys_hbm, ys_v)
    pltpu.sync_copy(zs_hbm, zs_v)
    pltpu.sync_copy(qx_hbm.at[pl.ds(base, QPW)], qx_v)
    pltpu.sync_copy(qy_hbm.at[pl.ds(base, QPW)], qy_v)
    pltpu.sync_copy(qz_hbm.at[pl.ds(base, QPW)], qz_v)
    lane = lax.iota(jnp.int32, L)
    sel15 = jnp.full((L,), L - 1, jnp.int32)
    shift_idx = jnp.maximum(lane - 1, 0)

    def prep(j, _):
        off = j * L
        x = xs_v[pl.ds(off, L)]
        y = ys_v[pl.ds(off, L)]
        z = zs_v[pl.ds(off, L)]
        s2_v[pl.ds(off, L)] = x * x + y * y + z * z
        xs_v[pl.ds(off, L)] = _round_bf16(x)
        ys_v[pl.ds(off, L)] = _round_bf16(y)
        zs_v[pl.ds(off, L)] = _round_bf16(z)
        return 0

    lax.fori_loop(0, NCHUNK, prep, 0)

    def per_pair(p, _):
        i0 = 2 * p
        g = i0 // L
        l0 = i0 % L
        sel0 = jnp.full((L,), l0, jnp.int32)
        sel1 = sel0 + 1
        qgx = qx_v[pl.ds(g * L, L)]
        qgy = qy_v[pl.ds(g * L, L)]
        qgz = qz_v[pl.ds(g * L, L)]
        axs = (_round_bf16(_splat(qgx, sel0)) * -2.0,
               _round_bf16(_splat(qgx, sel1)) * -2.0)
        ays = (_round_bf16(_splat(qgy, sel0)) * -2.0,
               _round_bf16(_splat(qgy, sel1)) * -2.0)
        azs = (_round_bf16(_splat(qgz, sel0)) * -2.0,
               _round_bf16(_splat(qgz, sel1)) * -2.0)

        def merge_chunk(td, ti, th, d, off):
            mask = d < th
            npos = plsc.all_reduce_population_count(mask)[0]

            def act(args):
                td, ti, _ = args

                def insert(a):
                    td, ti, _ = a
                    ffs = plsc.all_reduce_ffs(mask)
                    c = _splat(d, ffs)
                    ci = off + ffs
                    cnt_v = plsc.all_reduce_population_count(td <= c)
                    td_sh = _splat(td, shift_idx)
                    ti_sh = _splat(ti, shift_idx)
                    below = lane < cnt_v
                    at = lane == cnt_v
                    ntd = jnp.where(below, td, jnp.where(at, c, td_sh))
                    nti = jnp.where(below, ti, jnp.where(at, ci, ti_sh))
                    return ntd, nti, _splat(ntd, sel15)

                def full(a):
                    td, ti, _ = a
                    dd, di = plsc.sort_key_val(d, off + lane, descending=True)
                    keep = td <= dd
                    lo_d = jnp.where(keep, td, dd)
                    lo_i = jnp.where(keep, ti, di)
                    td2, ti2 = plsc.sort_key_val(lo_d, lo_i)
                    return td2, ti2, _splat(td2, sel15)

                return lax.cond(npos == 1, insert, full, (td, ti, th))

            return lax.cond(npos > 0, act, lambda a: a, (td, ti, th))

        def screen(td, ti, th, ds, off0):
            cs = [plsc.all_reduce_population_count(ds[u] < th)
                  for u in range(GROUP)]
            total = cs[0]
            for u in range(1, GROUP):
                total = total + cs[u]

            def single(a):
                td, ti, _ = a
                which = jnp.minimum(cs[0], 1) * 0
                for u in range(1, GROUP):
                    which = which + jnp.minimum(cs[u], 1) * u
                dsel = ds[0]
                for u in range(1, GROUP):
                    dsel = jnp.where(cs[u] > 0, ds[u], dsel)
                mask = dsel < th
                ffs = plsc.all_reduce_ffs(mask)
                c = _splat(dsel, ffs)
                ci = off0 + which * L + ffs
                cnt_v = plsc.all_reduce_population_count(td <= c)
                td_sh = _splat(td, shift_idx)
                ti_sh = _splat(ti, shift_idx)
                below = lane < cnt_v
                at = lane == cnt_v
                ntd = jnp.where(below, td, jnp.where(at, c, td_sh))
                nti = jnp.where(below, ti, jnp.where(at, ci, ti_sh))
                return ntd, nti, _splat(ntd, sel15)

            def multi(a):
                td, ti, th = a
                for u in range(GROUP):
                    td, ti, th = merge_chunk(td, ti, th, ds[u], off0 + u * L)
                return td, ti, th

            return lax.cond(total[0] == 1, single, multi, (td, ti, th))

        def group_step(jg, carry):
            td0, ti0, th0, td1, ti1, th1 = carry
            off0 = jg * (L * GROUP)
            d0s, d1s = [], []
            mv0 = mv1 = None
            for u in range(GROUP):
                off = off0 + u * L
                s2c = s2_v[pl.ds(off, L)]
                xc = xs_v[pl.ds(off, L)]
                yc = ys_v[pl.ds(off, L)]
                zc = zs_v[pl.ds(off, L)]
                d0 = s2c + axs[0] * xc + ays[0] * yc + azs[0] * zc
                d1 = s2c + axs[1] * xc + ays[1] * yc + azs[1] * zc
                d0s.append(d0)
                d1s.append(d1)
                mv0 = d0 if u == 0 else jnp.minimum(mv0, d0)
                mv1 = d1 if u == 0 else jnp.minimum(mv1, d1)

            def slow0(args):
                td, ti, th = args
                return screen(td, ti, th, d0s, off0)

            def slow1(args):
                td, ti, th = args
                return screen(td, ti, th, d1s, off0)

            hit0 = plsc.all_reduce_population_count(mv0 < th0)[0]
            td0, ti0, th0 = lax.cond(hit0 > 0, slow0, lambda a: a,
                                     (td0, ti0, th0))
            hit1 = plsc.all_reduce_population_count(mv1 < th1)[0]
            td1, ti1, th1 = lax.cond(hit1 > 0, slow1, lambda a: a,
                                     (td1, ti1, th1))
            return td0, ti0, th0, td1, ti1, th1

        init = (jnp.full((L,), jnp.inf, jnp.float32),
                jnp.zeros((L,), jnp.int32),
                jnp.full((L,), jnp.inf, jnp.float32)) * 2
        out = lax.fori_loop(0, NGROUP, group_step, init, unroll=2)
        plsc.store_scatter(idx_v, [lane, jnp.full((L,), i0, jnp.int32)],
                           out[1])
        plsc.store_scatter(idx_v, [lane, jnp.full((L,), i0 + 1, jnp.int32)],
                           out[4])
        return 0

    lax.fori_loop(0, QPW // 2, per_pair, 0)

    for k in range(K):
        pltpu.async_copy(feat_hbm.at[idx_v.at[k]], rows_v, sem).wait()
        pltpu.sync_copy(rows_v, out_hbm.at[pl.ds(k * M + base, QPW)])


BM = 256


def _tail_body(g_ref, gamma_ref, beta_ref, w_ref, o_ref):
    w = w_ref[:]
    gam = gamma_ref[:]
    bet = beta_ref[:]
    acc = jnp.full((BM, C_OUT), -jnp.inf, dtype=jnp.float32)
    for k in range(K):
        x = g_ref[k]
        mu = jnp.mean(x, axis=1, keepdims=True)
        xc = x - mu
        var = jnp.mean(xc * xc, axis=1, keepdims=True)
        xn = xc / jnp.sqrt(var + EPS) * gam + bet
        acc = jnp.maximum(acc, jnp.dot(xn, w, preferred_element_type=jnp.float32))
    o_ref[:] = acc


_tail = pl.pallas_call(
    _tail_body,
    grid=(M // BM,),
    in_specs=[
        pl.BlockSpec((K, BM, C_IN), lambda m: (0, m, 0)),
        pl.BlockSpec((1, C_IN), lambda m: (0, 0)),
        pl.BlockSpec((1, C_IN), lambda m: (0, 0)),
        pl.BlockSpec((C_IN, C_OUT), lambda m: (0, 0)),
    ],
    out_specs=pl.BlockSpec((BM, C_OUT), lambda m: (m, 0)),
    out_shape=jax.ShapeDtypeStruct((M, C_OUT), jnp.float32),
)


def kernel(src_xyz, src_feat, query_xyz, gamma, beta, W):
    s = src_xyz.T
    q = query_xyz.T
    grouped = _knn_gather(s[0], s[1], s[2], q[0], q[1], q[2], src_feat)
    g3 = grouped.reshape(K, M, C_IN)
    return _tail(g3, gamma.reshape(1, C_IN), beta.reshape(1, C_IN), W)
